# baseline (speedup 1.0000x reference)
import functools

import jax
import jax.numpy as jnp
from jax import lax
from jax.experimental import pallas as pl
from jax.experimental.pallas import tpu as pltpu
from jax.experimental.pallas import tpu_sc as plsc

_F = 64
_D = 1024
_NC = 2
_NS = 16
_NW = _NC * _NS
_GW = 128
_TP = 4096
_TPH = _TP // 2
_RB = 1024


def _tc_transpose(mt):
    v = mt.shape[1]
    nb = pl.cdiv(v, _TP)

    def body(m_ref, o_ref):
        m = m_ref[...]
        xt1 = jnp.transpose(m[:, :_TPH], (1, 0))
        xt2 = jnp.transpose(m[:, _TPH:], (1, 0))
        o_ref[...] = jnp.concatenate([xt1, xt2], axis=1)

    return pl.pallas_call(
        body,
        grid=(nb,),
        in_specs=[
            pl.BlockSpec((_F, _TP), lambda i: (0, i)),
        ],
        out_specs=pl.BlockSpec((_TPH, 2 * _F), lambda i: (i, 0)),
        out_shape=jax.ShapeDtypeStruct((nb * _TPH, 2 * _F), jnp.float32),
    )(mt)


def _sc_gather(table, idx):
    n = idx.shape[0]
    per_w = n // _NW
    kc = per_w // _GW
    assert per_w % _GW == 0 and n % _NW == 0
    mesh = plsc.VectorSubcoreMesh(core_axis_name="c", subcore_axis_name="s")

    @functools.partial(
        pl.kernel,
        mesh=mesh,
        out_type=jax.ShapeDtypeStruct((n, 2 * _F), jnp.float32),
        scratch_types=[
            pltpu.VMEM((per_w,), jnp.int32),
            pltpu.VMEM((_GW, 2 * _F), jnp.float32),
            pltpu.SemaphoreType.DMA,
        ],
    )
    def k(tab_hbm, idx_hbm, out_hbm, idx_v, rows_v, sem):
        wid = lax.axis_index("s") * _NC + lax.axis_index("c")
        base = wid * per_w
        pltpu.sync_copy(idx_hbm.at[pl.ds(base, per_w)], idx_v)

        @pl.loop(0, kc)
        def _(j):
            off = j * _GW
            pltpu.async_copy(
                tab_hbm.at[idx_v.at[pl.ds(off, _GW)]], rows_v, sem
            ).wait()
            pltpu.sync_copy(rows_v, out_hbm.at[pl.ds(base + off, _GW)])

    return k(table, idx)


def _tc_project(emb2, xi, w, b2d):
    n = emb2.shape[0]
    nb = n // _RB

    def body(e_ref, h_ref, w_ref, b_ref, o_ref):
        e = e_ref[...]
        sel = jnp.where(h_ref[...] == 1, e[:, _F:], e[:, :_F])
        o_ref[...] = lax.dot_general(
            sel, w_ref[...], (((1,), (1,)), ((), ())),
            preferred_element_type=jnp.float32,
        ) + b_ref[...]

    return pl.pallas_call(
        body,
        grid=(nb,),
        in_specs=[
            pl.BlockSpec((_RB, 2 * _F), lambda i: (i, 0)),
            pl.BlockSpec((_RB, 1), lambda i: (i, 0)),
            pl.BlockSpec((_D, _F), lambda i: (0, 0)),
            pl.BlockSpec((1, _D), lambda i: (0, 0)),
        ],
        out_specs=pl.BlockSpec((_RB, _D), lambda i: (i, 0)),
        out_shape=jax.ShapeDtypeStruct((n, _D), jnp.float32),
    )(emb2, xi, w, b2d)


def kernel(x, main_embed, W_proj, b_proj):
    bsz, seq = x.shape
    n = bsz * seq
    xf = x.reshape(n).astype(jnp.int32)
    tpair = _tc_transpose(main_embed.T)
    row = (lax.shift_right_logical(xf, 12) * _TPH) + (xf & (_TPH - 1))
    half = lax.shift_right_logical(xf, 11) & 1
    emb2 = _sc_gather(tpair, row)
    out = _tc_project(
        emb2, half.reshape(n, 1), W_proj, b_proj.reshape(1, _D)
    )
    return out.reshape(bsz, seq, _D)


# speedup vs baseline: 1.0916x; 1.0916x over previous
import functools

import jax
import jax.numpy as jnp
from jax import lax
from jax.experimental import pallas as pl
from jax.experimental.pallas import tpu as pltpu
from jax.experimental.pallas import tpu_sc as plsc

_F = 64
_D = 1024
_NC = 2
_NS = 16
_NW = _NC * _NS
_GW = 80
_TP = 4096
_RB = 1024
_K = 4


def _tc_transpose_pad(mt):
    v = mt.shape[1]
    nb = pl.cdiv(v, _TP)

    def body(m_ref, o_ref):
        xt = jnp.transpose(m_ref[...], (1, 0))
        o_ref[...] = jnp.concatenate(
            [xt, jnp.zeros((_TP, _F), jnp.float32)], axis=1
        )

    return pl.pallas_call(
        body,
        grid=(nb,),
        in_specs=[
            pl.BlockSpec((_F, _TP), lambda i: (0, i)),
        ],
        out_specs=pl.BlockSpec((_TP, 2 * _F), lambda i: (i, 0)),
        out_shape=jax.ShapeDtypeStruct((v, 2 * _F), jnp.float32),
    )(mt)


def _sc_gather(tpad, idx):
    n = idx.shape[0]
    per_w = n // _NW
    kc = per_w // _GW
    assert per_w % _GW == 0 and n % _NW == 0
    mesh = plsc.VectorSubcoreMesh(core_axis_name="c", subcore_axis_name="s")

    @functools.partial(
        pl.kernel,
        mesh=mesh,
        out_type=jax.ShapeDtypeStruct((n, 2 * _F), jnp.float32),
        scratch_types=[
            pltpu.VMEM((per_w,), jnp.int32),
            pltpu.VMEM((_GW, 2 * _F), jnp.float32),
            pltpu.SemaphoreType.DMA,
        ],
    )
    def k(tab_hbm, idx_hbm, out_hbm, idx_v, rows_v, sem):
        wid = lax.axis_index("s") * _NC + lax.axis_index("c")
        base = wid * per_w
        pltpu.sync_copy(idx_hbm.at[pl.ds(base, per_w)], idx_v)

        @pl.loop(0, kc)
        def _(j):
            off = j * _GW
            pltpu.async_copy(
                tab_hbm.at[idx_v.at[pl.ds(off, _GW)]], rows_v, sem
            ).wait()
            pltpu.sync_copy(rows_v, out_hbm.at[pl.ds(base + off, _GW)])

    return k(tpad, idx)


def _tc_project_chunk(acc, emb2, w, b2d, n_total, blk_off):
    nc = emb2.shape[0]
    nb = nc // _RB

    def body(*refs):
        e_ref, w_ref, b_ref, o_ref = refs[-4:]
        o_ref[...] = lax.dot_general(
            e_ref[:, :_F], w_ref[...], (((1,), (1,)), ((), ())),
            preferred_element_type=jnp.float32,
        ) + b_ref[...]

    specs = [
        pl.BlockSpec((_RB, 2 * _F), lambda i: (i, 0)),
        pl.BlockSpec((_D, _F), lambda i: (0, 0)),
        pl.BlockSpec((1, _D), lambda i: (0, 0)),
    ]
    out_spec = pl.BlockSpec((_RB, _D), lambda i: (blk_off + i, 0))
    out_shape = jax.ShapeDtypeStruct((n_total, _D), jnp.float32)
    if acc is None:
        return pl.pallas_call(
            body,
            grid=(nb,),
            in_specs=specs,
            out_specs=out_spec,
            out_shape=out_shape,
        )(emb2, w, b2d)
    return pl.pallas_call(
        body,
        grid=(nb,),
        in_specs=[pl.BlockSpec(memory_space=pltpu.MemorySpace.HBM)] + specs,
        out_specs=out_spec,
        out_shape=out_shape,
        input_output_aliases={0: 0},
    )(acc, emb2, w, b2d)


def kernel(x, main_embed, W_proj, b_proj):
    bsz, seq = x.shape
    n = bsz * seq
    nc = n // _K
    xf = x.reshape(n).astype(jnp.int32)
    tpad = _tc_transpose_pad(main_embed.T)
    b2d = b_proj.reshape(1, _D)
    embs = [
        _sc_gather(tpad, lax.dynamic_slice_in_dim(xf, k * nc, nc))
        for k in range(_K)
    ]
    out = None
    for k in range(_K):
        out = _tc_project_chunk(
            out, embs[k], W_proj, b2d, n, k * (nc // _RB)
        )
    return out.reshape(bsz, seq, _D)


# speedup vs baseline: 1.2019x; 1.1010x over previous
import functools

import jax
import jax.numpy as jnp
from jax import lax
from jax.experimental import pallas as pl
from jax.experimental.pallas import tpu as pltpu
from jax.experimental.pallas import tpu_sc as plsc

_F = 64
_D = 1024
_NC = 2
_NS = 16
_NW = _NC * _NS
_GW = 80
_TP = 8192
_RB = 1024
_K = 4


def _tc_transpose_pad(mt):
    v = mt.shape[1]
    nb = pl.cdiv(v, _TP)

    def body(m_ref, o_ref):
        xt = jnp.transpose(m_ref[...], (1, 0))
        o_ref[...] = jnp.concatenate(
            [xt, jnp.zeros((_TP, _F), jnp.float32)], axis=1
        )

    return pl.pallas_call(
        body,
        grid=(nb,),
        in_specs=[
            pl.BlockSpec((_F, _TP), lambda i: (0, i)),
        ],
        out_specs=pl.BlockSpec((_TP, 2 * _F), lambda i: (i, 0)),
        out_shape=jax.ShapeDtypeStruct((v, 2 * _F), jnp.float32),
    )(mt)


def _sc_gather(tpad, idx):
    n = idx.shape[0]
    per_w = n // _NW
    kc = per_w // _GW
    assert per_w % _GW == 0 and n % _NW == 0
    mesh = plsc.VectorSubcoreMesh(core_axis_name="c", subcore_axis_name="s")

    @functools.partial(
        pl.kernel,
        mesh=mesh,
        out_type=jax.ShapeDtypeStruct((n, 2 * _F), jnp.float32),
        scratch_types=[
            pltpu.VMEM((per_w,), jnp.int32),
            pltpu.VMEM((_GW, 2 * _F), jnp.float32),
            pltpu.SemaphoreType.DMA,
        ],
    )
    def k(tab_hbm, idx_hbm, out_hbm, idx_v, rows_v, sem):
        wid = lax.axis_index("s") * _NC + lax.axis_index("c")
        base = wid * per_w
        pltpu.sync_copy(idx_hbm.at[pl.ds(base, per_w)], idx_v)

        @pl.loop(0, kc)
        def _(j):
            off = j * _GW
            pltpu.async_copy(
                tab_hbm.at[idx_v.at[pl.ds(off, _GW)]], rows_v, sem
            ).wait()
            pltpu.sync_copy(rows_v, out_hbm.at[pl.ds(base + off, _GW)])

    return k(tpad, idx)


def _tc_project_chunk(acc, emb2, w, b2d, n_total, blk_off):
    nc = emb2.shape[0]
    nb = nc // _RB

    def body(*refs):
        e_ref, w_ref, b_ref, o_ref = refs[-4:]
        o_ref[...] = lax.dot_general(
            e_ref[:, :_F], w_ref[...], (((1,), (1,)), ((), ())),
            preferred_element_type=jnp.float32,
        ) + b_ref[...]

    specs = [
        pl.BlockSpec((_RB, 2 * _F), lambda i: (i, 0)),
        pl.BlockSpec((_D, _F), lambda i: (0, 0)),
        pl.BlockSpec((1, _D), lambda i: (0, 0)),
    ]
    out_spec = pl.BlockSpec((_RB, _D), lambda i: (blk_off + i, 0))
    out_shape = jax.ShapeDtypeStruct((n_total, _D), jnp.float32)
    if acc is None:
        return pl.pallas_call(
            body,
            grid=(nb,),
            in_specs=specs,
            out_specs=out_spec,
            out_shape=out_shape,
        )(emb2, w, b2d)
    return pl.pallas_call(
        body,
        grid=(nb,),
        in_specs=[pl.BlockSpec(memory_space=pltpu.MemorySpace.HBM)] + specs,
        out_specs=out_spec,
        out_shape=out_shape,
        input_output_aliases={0: 0},
    )(acc, emb2, w, b2d)


def kernel(x, main_embed, W_proj, b_proj):
    bsz, seq = x.shape
    n = bsz * seq
    nc = n // _K
    xf = x.reshape(n).astype(jnp.int32)
    tpad = _tc_transpose_pad(main_embed.T)
    b2d = b_proj.reshape(1, _D)
    embs = [
        _sc_gather(tpad, lax.dynamic_slice_in_dim(xf, k * nc, nc))
        for k in range(_K)
    ]
    out = None
    for k in range(_K):
        out = _tc_project_chunk(
            out, embs[k], W_proj, b2d, n, k * (nc // _RB)
        )
    return out.reshape(bsz, seq, _D)
